# Initial kernel scaffold; baseline (speedup 1.0000x reference)
import jax
import jax.numpy as jnp
from jax.experimental import pallas as pl


def kernel(action_probs, demo_gender, demo_age):
    raise NotImplementedError("write your pallas kernel here")


# speedup vs baseline: 3.1926x; 3.1926x over previous
import functools

import jax
import jax.numpy as jnp
from jax import lax
from jax.experimental import pallas as pl
from jax.experimental.pallas import tpu as pltpu
from jax.experimental.pallas import tpu_sc as plsc

BATCH = 16384
NUM_ACTIONS = 1000
NSENS = 16
NGROUPS = 10
LAMBDA_FAIRNESS = 0.01

NW = 32
RPW = BATCH // NW
CHUNK = 16
NCHUNK = RPW // CHUNK


def _sc_partials(x_hbm, g_hbm, a_hbm, sums_hbm, cnt_hbm, xv, gv, av, accv, cntv):
    wid = lax.axis_index("s") * 2 + lax.axis_index("c")
    base = wid * RPW
    pltpu.sync_copy(x_hbm.at[pl.ds(base * NSENS, RPW * NSENS)], xv)
    pltpu.sync_copy(g_hbm.at[pl.ds(base, RPW)], gv)
    pltpu.sync_copy(a_hbm.at[pl.ds(base, RPW)], av)

    lane = lax.iota(jnp.int32, 16)
    zero16 = jnp.zeros((16,), jnp.float32)
    ones16 = jnp.ones((16,), jnp.float32)
    for r in range(16):
        accv[pl.ds(r * 16, 16)] = zero16
        cntv[pl.ds(r * 16, 16)] = zero16

    def chunk_body(c, carry):
        g16 = gv[pl.ds(c * CHUNK, 16)]
        a16 = av[pl.ds(c * CHUNK, 16)]
        plsc.addupdate_scatter(cntv, [g16 * 16 + lane], ones16)
        plsc.addupdate_scatter(cntv, [(a16 + 2) * 16 + lane], ones16)
        for i in range(CHUNK):
            row = c * CHUNK + i
            rsplat = jnp.full((16,), row, jnp.int32)
            xrow = xv[pl.ds(row * NSENS, 16)]
            gsp = plsc.load_gather(gv, [rsplat])
            asp = plsc.load_gather(av, [rsplat])
            plsc.addupdate_scatter(accv, [gsp * 16 + lane], xrow)
            plsc.addupdate_scatter(accv, [(asp + 2) * 16 + lane], xrow)
        return carry

    lax.fori_loop(0, NCHUNK, chunk_body, jnp.int32(0))
    pltpu.sync_copy(accv, sums_hbm.at[wid])
    pltpu.sync_copy(cntv, cnt_hbm.at[wid])


def _pairmask():
    rj = lax.broadcasted_iota(jnp.int32, (16, 16), 0)
    ck = lax.broadcasted_iota(jnp.int32, (16, 16), 1)
    same = jnp.logical_or(
        jnp.logical_and(rj < 2, ck < 2),
        jnp.logical_and(jnp.logical_and(rj >= 2, rj < 10),
                        jnp.logical_and(ck >= 2, ck < 10)))
    return jnp.logical_and(rj > ck, same).astype(jnp.float32)


def _epilogue(s_ref, c_ref, out_ref):
    sums = jnp.sum(s_ref[...], axis=0)
    counts = jnp.sum(jnp.sum(c_ref[...], axis=0), axis=1,
                     keepdims=True)
    present = (counts > 0.0).astype(jnp.float32)
    safe = jnp.where(counts > 0.0, counts, 1.0)
    means = sums / safe
    both = lax.dot_general(
        present, present, (((1,), (1,)), ((), ())),
        preferred_element_type=jnp.float32)
    pm = _pairmask()
    ncomp = float(NSENS) * jnp.sum(pm * both)
    total = jnp.float32(0.0)
    for k in range(NGROUPS):
        d = jnp.abs(means - means[k:k + 1, :])
        s = jnp.sum(d, axis=1, keepdims=True)
        total = total + jnp.sum(s * pm[:, k:k + 1] * both[:, k:k + 1])
    result = jnp.where(
        ncomp > 0.0,
        LAMBDA_FAIRNESS * total / jnp.maximum(ncomp, 1.0),
        0.0)
    out_ref[0, 0] = result


@jax.jit
def kernel(action_probs, demo_gender, demo_age):
    mesh = plsc.VectorSubcoreMesh(core_axis_name="c", subcore_axis_name="s")
    sums, cnt = pl.kernel(
        _sc_partials,
        mesh=mesh,
        compiler_params=pltpu.CompilerParams(needs_layout_passes=False),
        out_type=[
            jax.ShapeDtypeStruct((NW, 256), jnp.float32),
            jax.ShapeDtypeStruct((NW, 256), jnp.float32),
        ],
        scratch_types=[
            pltpu.VMEM((RPW * NSENS,), jnp.float32),
            pltpu.VMEM((RPW,), jnp.int32),
            pltpu.VMEM((RPW,), jnp.int32),
            pltpu.VMEM((256,), jnp.float32),
            pltpu.VMEM((256,), jnp.float32),
        ],
    )(action_probs[:, :NSENS].reshape(-1), demo_gender, demo_age)
    out = pl.pallas_call(
        _epilogue,
        out_specs=pl.BlockSpec(memory_space=pltpu.SMEM),
        out_shape=jax.ShapeDtypeStruct((1, 1), jnp.float32),
    )(sums.reshape(NW, 16, 16), cnt.reshape(NW, 16, 16))
    return out[0, 0]
